# baseline (speedup 1.0000x reference)
import functools

import jax
import jax.numpy as jnp
from jax import lax
from jax.experimental import pallas as pl
from jax.experimental.pallas import tpu as pltpu
from jax.experimental.pallas import tpu_sc as plsc

N = 10000
D_IN = 128
D_HID = 256
E = 320000

NC, NS, L = 2, 16, 16
NW = NC * NS
N_PAD = 10240
ROWS_PER_TILE = N_PAD // NS
E_PAD = 327680
E_PER_TILE = E_PAD // NW
W_WIN = 128
N_WIN = E_PER_TILE // W_WIN

_MESH = plsc.VectorSubcoreMesh(core_axis_name="c", subcore_axis_name="s")


def _zero_fill(ref, n16):
    def body(i, _):
        ref[pl.ds(i * 16, 16)] = jnp.zeros((16,), jnp.float32)
        return 0
    lax.fori_loop(0, n16, body, 0)


@functools.partial(
    pl.kernel,
    out_type=jax.ShapeDtypeStruct((NC, N_PAD), jnp.float32),
    mesh=_MESH,
    scratch_types=[
        pltpu.VMEM((N_WIN, W_WIN), jnp.int32),
        pltpu.VMEM((W_WIN,), jnp.float32),
        pltpu.VMEM((ROWS_PER_TILE,), jnp.float32),
        pltpu.VMEM_SHARED((N_PAD,), jnp.float32),
    ],
)
def _deg_kernel(dst_hbm, deg_hbm, didx, ones_v, zero_v, deg_sh):
    c = lax.axis_index("c")
    s = lax.axis_index("s")
    wid = c * NS + s

    def fill_ones(i, _):
        ones_v[pl.ds(i * 16, 16)] = jnp.ones((16,), jnp.float32)
        return 0
    lax.fori_loop(0, W_WIN // 16, fill_ones, 0)
    _zero_fill(zero_v, ROWS_PER_TILE // 16)

    pltpu.sync_copy(zero_v, deg_sh.at[pl.ds(s * ROWS_PER_TILE, ROWS_PER_TILE)])
    plsc.subcore_barrier()

    pltpu.sync_copy(dst_hbm.at[wid], didx)

    def body(j, _):
        pltpu.sync_copy(ones_v, deg_sh.at[didx.at[j]], add=True)
        return 0
    lax.fori_loop(0, N_WIN, body, 0)

    plsc.subcore_barrier()
    sl = pl.ds(s * ROWS_PER_TILE, ROWS_PER_TILE)
    pltpu.sync_copy(deg_sh.at[sl], deg_hbm.at[c, sl])


def _scale_body(degA, degB, x, dinv_o, xs_o):
    d = degA[...] + degB[...] + 1.0
    dv = lax.rsqrt(d)
    dinv_o[...] = dv
    xs_o[...] = x[...] * dv[:, None]


def _scale_call(degA, degB, x):
    blk = 1024
    grid = N_PAD // blk
    return pl.pallas_call(
        _scale_body,
        grid=(grid,),
        in_specs=[
            pl.BlockSpec((blk,), lambda i: (i,)),
            pl.BlockSpec((blk,), lambda i: (i,)),
            pl.BlockSpec((blk, D_IN), lambda i: (i, 0)),
        ],
        out_specs=[
            pl.BlockSpec((blk,), lambda i: (i,)),
            pl.BlockSpec((blk, D_IN), lambda i: (i, 0)),
        ],
        out_shape=[
            jax.ShapeDtypeStruct((N_PAD,), jnp.float32),
            jax.ShapeDtypeStruct((N_PAD, D_IN), jnp.float32),
        ],
    )(degA, degB, x)


@functools.partial(
    pl.kernel,
    out_type=jax.ShapeDtypeStruct((NC, N_PAD, D_IN), jnp.float32),
    mesh=_MESH,
    scratch_types=[
        pltpu.VMEM((N_WIN // 2, W_WIN), jnp.int32),
        pltpu.VMEM((N_WIN // 2, W_WIN), jnp.int32),
        pltpu.VMEM((2, W_WIN, D_IN), jnp.float32),
        pltpu.VMEM((16, D_IN), jnp.float32),
        pltpu.VMEM_SHARED((N_PAD, D_IN), jnp.float32),
        pltpu.SemaphoreType.DMA,
        pltpu.SemaphoreType.DMA,
    ],
)
def _prop_kernel(src_hbm, dst_hbm, xs_hbm, p_hbm,
                 sidx, didx, buf, zrow, p_sh, gsem0, gsem1):
    c = lax.axis_index("c")
    s = lax.axis_index("s")
    wid = c * NS + s

    def zfill(i, _):
        zrow[i // 8, pl.ds((i % 8) * 16, 16)] = jnp.zeros((16,), jnp.float32)
        return 0
    lax.fori_loop(0, 16 * (D_IN // 16), zfill, 0)
    row0 = s * ROWS_PER_TILE

    def zbody(k, _):
        pltpu.sync_copy(zrow, p_sh.at[pl.ds(row0 + k * 16, 16)])
        return 0
    lax.fori_loop(0, ROWS_PER_TILE // 16, zbody, 0)
    plsc.subcore_barrier()

    gsems = (gsem0, gsem1)
    nw = N_WIN // 2
    for half in range(2):
        pltpu.sync_copy(src_hbm.at[wid, pl.ds(half * nw, nw)], sidx)
        pltpu.sync_copy(dst_hbm.at[wid, pl.ds(half * nw, nw)], didx)
        pltpu.async_copy(xs_hbm.at[sidx.at[0]], buf.at[0], gsem0)

        def body(k, _):
            for b in range(2):
                j = k * 2 + b
                o = 1 - b
                pltpu.make_async_copy(xs_hbm.at[sidx.at[j]], buf.at[b],
                                      gsems[b]).wait()

                @pl.when(j + 1 < nw)
                def _():
                    pltpu.async_copy(xs_hbm.at[sidx.at[j + 1]], buf.at[o],
                                     gsems[o])
                pltpu.sync_copy(buf.at[b], p_sh.at[didx.at[j]], add=True)
            return 0
        lax.fori_loop(0, nw // 2, body, 0)

    plsc.subcore_barrier()
    sl = pl.ds(row0, ROWS_PER_TILE)
    pltpu.sync_copy(p_sh.at[sl], p_hbm.at[c, sl])


def _dense_body(pA, pB, xs, dinv, W1, b1, W2t, zs_o):
    dv = dinv[...]
    u = (pA[...] + pB[...] + xs[...]) * dv[:, None]
    h1 = jnp.dot(u, W1[...], preferred_element_type=jnp.float32,
                 precision=jax.lax.Precision.HIGHEST)
    h1 = jnp.maximum(h1 + b1[...], 0.0)
    z = jnp.sum(h1 * W2t[...], axis=1)
    zs_o[...] = z * dv


def _dense_call(pA, pB, xs, dinv, W1, b1, W2t):
    blk = 512
    grid = N_PAD // blk
    return pl.pallas_call(
        _dense_body,
        grid=(grid,),
        in_specs=[
            pl.BlockSpec((blk, D_IN), lambda i: (i, 0)),
            pl.BlockSpec((blk, D_IN), lambda i: (i, 0)),
            pl.BlockSpec((blk, D_IN), lambda i: (i, 0)),
            pl.BlockSpec((blk,), lambda i: (i,)),
            pl.BlockSpec((D_IN, D_HID), lambda i: (0, 0)),
            pl.BlockSpec((1, D_HID), lambda i: (0, 0)),
            pl.BlockSpec((1, D_HID), lambda i: (0, 0)),
        ],
        out_specs=pl.BlockSpec((blk,), lambda i: (i,)),
        out_shape=jax.ShapeDtypeStruct((N_PAD,), jnp.float32),
    )(pA, pB, xs, dinv, W1, b1, W2t)


@functools.partial(
    pl.kernel,
    out_type=jax.ShapeDtypeStruct((NC, N_PAD), jnp.float32),
    mesh=_MESH,
    scratch_types=[
        pltpu.VMEM((E_PER_TILE,), jnp.int32),
        pltpu.VMEM((N_WIN, W_WIN), jnp.int32),
        pltpu.VMEM((N_PAD,), jnp.float32),
        pltpu.VMEM((W_WIN,), jnp.float32),
        pltpu.VMEM((ROWS_PER_TILE,), jnp.float32),
        pltpu.VMEM_SHARED((N_PAD,), jnp.float32),
    ],
    compiler_params=pltpu.CompilerParams(needs_layout_passes=False),
)
def _sprop_kernel(srcf_hbm, dst_hbm, zs_hbm, o_hbm,
                  sidx, didx, zs_v, vals, zero_v, o_sh):
    c = lax.axis_index("c")
    s = lax.axis_index("s")
    wid = c * NS + s

    _zero_fill(zero_v, ROWS_PER_TILE // 16)
    pltpu.sync_copy(zero_v, o_sh.at[pl.ds(s * ROWS_PER_TILE, ROWS_PER_TILE)])
    plsc.subcore_barrier()

    pltpu.sync_copy(srcf_hbm.at[wid], sidx)
    pltpu.sync_copy(dst_hbm.at[wid], didx)
    pltpu.sync_copy(zs_hbm, zs_v)

    def body(j, _):
        for k in range(W_WIN // 16):
            i16 = sidx[pl.ds(j * W_WIN + k * 16, 16)]
            vals[pl.ds(k * 16, 16)] = plsc.load_gather(zs_v, [i16])
        pltpu.sync_copy(vals, o_sh.at[didx.at[j]], add=True)
        return 0
    lax.fori_loop(0, N_WIN, body, 0)

    plsc.subcore_barrier()
    sl = pl.ds(s * ROWS_PER_TILE, ROWS_PER_TILE)
    pltpu.sync_copy(o_sh.at[sl], o_hbm.at[c, sl])


def _comb_body(oA, oB, zs, dinv, b2, out_o):
    out_o[...] = (oA[...] + oB[...] + zs[...]) * dinv[...] + b2[0]


def _comb_call(oA, oB, zs, dinv, b2):
    blk = 1024
    grid = N_PAD // blk
    return pl.pallas_call(
        _comb_body,
        grid=(grid,),
        in_specs=[
            pl.BlockSpec((blk,), lambda i: (i,)),
            pl.BlockSpec((blk,), lambda i: (i,)),
            pl.BlockSpec((blk,), lambda i: (i,)),
            pl.BlockSpec((blk,), lambda i: (i,)),
            pl.BlockSpec(memory_space=pltpu.SMEM),
        ],
        out_specs=pl.BlockSpec((blk,), lambda i: (i,)),
        out_shape=jax.ShapeDtypeStruct((N_PAD,), jnp.float32),
    )(oA, oB, zs, dinv, b2)


def kernel(x, edge_index, W1, b1, W2, b2):
    ei = edge_index.astype(jnp.int32)
    pad = jnp.full((E_PAD - E,), N, jnp.int32)
    src = jnp.concatenate([ei[0], pad])
    dst = jnp.concatenate([ei[1], pad])
    src_w = src.reshape(NW, N_WIN, W_WIN)
    dst_w = dst.reshape(NW, N_WIN, W_WIN)
    src_f = src.reshape(NW, E_PER_TILE)
    x_pad = jnp.pad(x, ((0, N_PAD - N), (0, 0)))

    deg2 = _deg_kernel(dst_w)
    dinv, xs = _scale_call(deg2[0], deg2[1], x_pad)
    p2 = _prop_kernel(src_w, dst_w, xs)
    zs = _dense_call(p2[0], p2[1], xs, dinv, W1,
                     b1.reshape(1, D_HID), W2.reshape(1, D_HID))
    o2 = _sprop_kernel(src_f, dst_w, zs)
    out = _comb_call(o2[0], o2[1], zs, dinv, b2)
    return out[:N, None]


# speedup vs baseline: 2.3822x; 2.3822x over previous
import functools

import jax
import jax.numpy as jnp
from jax import lax
from jax.experimental import pallas as pl
from jax.experimental.pallas import tpu as pltpu
from jax.experimental.pallas import tpu_sc as plsc

N = 10000
D_IN = 128
D_HID = 256
E = 320000

NC, NS = 2, 16
NW = NC * NS
N_PAD = 10240
ROWS_PER_TILE = N_PAD // NS

W3 = 40
GW3 = E // W3
T3 = GW3 // NW
W15 = 128
GW15 = E // W15
T15_LO = GW15 // NW
T15_EXTRA = GW15 - NW * T15_LO

_MESH = plsc.VectorSubcoreMesh(core_axis_name="c", subcore_axis_name="s")
_SC_PARAMS = pltpu.CompilerParams(use_tc_tiling_on_sc=False)


def _zero_fill(ref, n16):
    def body(i, _):
        ref[pl.ds(i * 16, 16)] = jnp.zeros((16,), jnp.float32)
        return 0
    lax.fori_loop(0, n16, body, 0)


def _ring(src_hbm, sidx, didx, buf, dst_sh, gsems, ssems, cnt):
    pltpu.async_copy(src_hbm.at[sidx.at[0]], buf.at[0], gsems[0])
    pltpu.async_copy(src_hbm.at[sidx.at[1]], buf.at[1], gsems[1])

    def one(j, b):
        f = (b + 2) % 4
        pltpu.make_async_copy(src_hbm.at[sidx.at[j]], buf.at[b],
                              gsems[b]).wait()
        pltpu.async_copy(buf.at[b], dst_sh.at[didx.at[j]], ssems[b], add=True)

        @pl.when(j >= 2)
        def _():
            pltpu.make_async_copy(buf.at[f], dst_sh.at[didx.at[j - 2]],
                                  ssems[f]).wait()

        @pl.when(j + 2 < cnt)
        def _():
            pltpu.async_copy(src_hbm.at[sidx.at[j + 2]], buf.at[f], gsems[f])

    def body(k, _):
        for b in range(4):
            one(k * 4 + b, b)
        return 0
    lax.fori_loop(0, cnt // 4, body, 0)
    for j in range(4 * (cnt // 4), cnt):
        one(j, j % 4)
    for j in range(cnt - 2, cnt):
        pltpu.make_async_copy(buf.at[j % 4], dst_sh.at[didx.at[j]],
                              ssems[j % 4]).wait()


@functools.partial(
    pl.kernel,
    out_type=jax.ShapeDtypeStruct((NC, N_PAD), jnp.float32),
    mesh=_MESH,
    scratch_types=[
        pltpu.VMEM((T15_LO + 1, W15), jnp.int32),
        pltpu.VMEM((W15,), jnp.float32),
        pltpu.VMEM((ROWS_PER_TILE,), jnp.float32),
        pltpu.VMEM_SHARED((N_PAD,), jnp.float32),
    ],
    compiler_params=_SC_PARAMS,
)
def _deg_kernel(ei15_hbm, deg_hbm, didx, ones_v, zero_v, deg_sh):
    c = lax.axis_index("c")
    s = lax.axis_index("s")
    wid = c * NS + s

    def fill_ones(i, _):
        ones_v[pl.ds(i * 16, 16)] = jnp.ones((16,), jnp.float32)
        return 0
    lax.fori_loop(0, W15 // 16, fill_ones, 0)
    _zero_fill(zero_v, ROWS_PER_TILE // 16)

    pltpu.sync_copy(zero_v, deg_sh.at[pl.ds(s * ROWS_PER_TILE, ROWS_PER_TILE)])
    plsc.subcore_barrier()

    def scatter_ones(base, cnt):
        pltpu.sync_copy(ei15_hbm.at[1, pl.ds(base, cnt)],
                        didx.at[pl.ds(0, cnt)])

        def body(j, _):
            pltpu.sync_copy(ones_v, deg_sh.at[didx.at[j]], add=True)
            return 0
        lax.fori_loop(0, cnt, body, 0)

    @pl.when(wid < T15_EXTRA)
    def _():
        scatter_ones((T15_LO + 1) * wid, T15_LO + 1)

    @pl.when(wid >= T15_EXTRA)
    def _():
        scatter_ones(T15_LO * wid + T15_EXTRA, T15_LO)

    plsc.subcore_barrier()
    sl = pl.ds(s * ROWS_PER_TILE, ROWS_PER_TILE)
    pltpu.sync_copy(deg_sh.at[sl], deg_hbm.at[c, sl])


def _scale_body(deg2, x, dinv_o, xs_o):
    d = deg2[0, :N] + deg2[1, :N] + 1.0
    dv = lax.rsqrt(d)
    dinv_o[...] = dv
    xs_o[...] = x[...] * dv[:, None]


def _scale_call(deg2, x):
    return pl.pallas_call(
        _scale_body,
        out_shape=[
            jax.ShapeDtypeStruct((N,), jnp.float32),
            jax.ShapeDtypeStruct((N, D_IN), jnp.float32),
        ],
    )(deg2, x)


@functools.partial(
    pl.kernel,
    out_type=jax.ShapeDtypeStruct((NC, N_PAD, D_IN), jnp.float32),
    mesh=_MESH,
    scratch_types=[
        pltpu.VMEM((128, W3), jnp.int32),
        pltpu.VMEM((128, W3), jnp.int32),
        pltpu.VMEM((4, W3, D_IN), jnp.float32),
        pltpu.VMEM((16, D_IN), jnp.float32),
        pltpu.VMEM_SHARED((N_PAD, D_IN), jnp.float32),
        [pltpu.SemaphoreType.DMA] * 4,
        [pltpu.SemaphoreType.DMA] * 4,
    ],
    compiler_params=_SC_PARAMS,
)
def _prop_kernel(ei3_hbm, xs_hbm, p_hbm,
                 sidx, didx, buf, zrow, p_sh, gsems, ssems):
    c = lax.axis_index("c")
    s = lax.axis_index("s")
    wid = c * NS + s

    def zfill(i, _):
        zrow[i // 8, pl.ds((i % 8) * 16, 16)] = jnp.zeros((16,), jnp.float32)
        return 0
    lax.fori_loop(0, 16 * (D_IN // 16), zfill, 0)
    row0 = s * ROWS_PER_TILE

    def zbody(k, _):
        pltpu.sync_copy(zrow, p_sh.at[pl.ds(row0 + k * 16, 16)])
        return 0
    lax.fori_loop(0, ROWS_PER_TILE // 16, zbody, 0)
    plsc.subcore_barrier()

    off = 0
    for cnt in (128, T3 - 128):
        w0 = wid * T3 + off
        pltpu.sync_copy(ei3_hbm.at[0, pl.ds(w0, cnt)], sidx.at[pl.ds(0, cnt)])
        pltpu.sync_copy(ei3_hbm.at[1, pl.ds(w0, cnt)], didx.at[pl.ds(0, cnt)])
        _ring(xs_hbm, sidx, didx, buf, p_sh, gsems, ssems, cnt)
        off += cnt

    plsc.subcore_barrier()
    sl = pl.ds(row0, ROWS_PER_TILE)
    pltpu.sync_copy(p_sh.at[sl], p_hbm.at[c, sl])


def _dense_body(p2, xs, dinv, W1, b1, W2t, zs_o):
    dv = dinv[...]
    u = (p2[0, :N] + p2[1, :N] + xs[...]) * dv[:, None]
    h1 = jnp.dot(u, W1[...], preferred_element_type=jnp.float32,
                 precision=jax.lax.Precision.HIGHEST)
    h1 = jnp.maximum(h1 + b1[...], 0.0)
    z = jnp.sum(h1 * W2t[...], axis=1)
    zs_o[...] = z * dv


def _dense_call(p2, xs, dinv, W1, b1, W2t):
    return pl.pallas_call(
        _dense_body,
        out_shape=jax.ShapeDtypeStruct((N,), jnp.float32),
    )(p2, xs, dinv, W1, b1, W2t)


@functools.partial(
    pl.kernel,
    out_type=jax.ShapeDtypeStruct((NC, N_PAD), jnp.float32),
    mesh=_MESH,
    scratch_types=[
        pltpu.VMEM((T15_LO + 1, W15), jnp.int32),
        pltpu.VMEM((T15_LO + 1, W15), jnp.int32),
        pltpu.VMEM((4, W15), jnp.float32),
        pltpu.VMEM((ROWS_PER_TILE,), jnp.float32),
        pltpu.VMEM_SHARED((N_PAD,), jnp.float32),
        [pltpu.SemaphoreType.DMA] * 4,
        [pltpu.SemaphoreType.DMA] * 4,
    ],
    compiler_params=_SC_PARAMS,
)
def _sprop_kernel(ei15_hbm, zs_hbm, o_hbm,
                  sidx, didx, buf, zero_v, o_sh, gsems, ssems):
    c = lax.axis_index("c")
    s = lax.axis_index("s")
    wid = c * NS + s

    _zero_fill(zero_v, ROWS_PER_TILE // 16)
    pltpu.sync_copy(zero_v, o_sh.at[pl.ds(s * ROWS_PER_TILE, ROWS_PER_TILE)])
    plsc.subcore_barrier()

    def run(base, cnt):
        pltpu.sync_copy(ei15_hbm.at[0, pl.ds(base, cnt)],
                        sidx.at[pl.ds(0, cnt)])
        pltpu.sync_copy(ei15_hbm.at[1, pl.ds(base, cnt)],
                        didx.at[pl.ds(0, cnt)])
        _ring(zs_hbm, sidx, didx, buf, o_sh, gsems, ssems, cnt)

    @pl.when(wid < T15_EXTRA)
    def _():
        run((T15_LO + 1) * wid, T15_LO + 1)

    @pl.when(wid >= T15_EXTRA)
    def _():
        run(T15_LO * wid + T15_EXTRA, T15_LO)

    plsc.subcore_barrier()
    sl = pl.ds(s * ROWS_PER_TILE, ROWS_PER_TILE)
    pltpu.sync_copy(o_sh.at[sl], o_hbm.at[c, sl])


def _comb_body(o2, zs, dinv, b2, out_o):
    out_o[...] = (o2[0, :N] + o2[1, :N] + zs[...]) * dinv[...] + b2[0]


def _comb_call(o2, zs, dinv, b2):
    return pl.pallas_call(
        _comb_body,
        in_specs=[
            pl.BlockSpec(),
            pl.BlockSpec(),
            pl.BlockSpec(),
            pl.BlockSpec(memory_space=pltpu.SMEM),
        ],
        out_specs=pl.BlockSpec(),
        out_shape=jax.ShapeDtypeStruct((N,), jnp.float32),
    )(o2, zs, dinv, b2)


def kernel(x, edge_index, W1, b1, W2, b2):
    ei = edge_index.astype(jnp.int32)
    ei3 = ei.reshape(2, GW3, W3)
    ei15 = ei.reshape(2, GW15, W15)

    deg2 = _deg_kernel(ei15)
    dinv, xs = _scale_call(deg2, x)
    p2 = _prop_kernel(ei3, xs)
    zs = _dense_call(p2, xs, dinv, W1,
                     b1.reshape(1, D_HID), W2.reshape(1, D_HID))
    o2 = _sprop_kernel(ei15, zs)
    out = _comb_call(o2, zs, dinv, b2)
    return out[:, None]


# speedup vs baseline: 2.7050x; 1.1355x over previous
import functools

import jax
import jax.numpy as jnp
from jax import lax
from jax.experimental import pallas as pl
from jax.experimental.pallas import tpu as pltpu
from jax.experimental.pallas import tpu_sc as plsc

N = 10000
D_IN = 128
D_HID = 256
E = 320000

NC, NS = 2, 16
NW = NC * NS
N_PAD = 10240
ROWS_PER_TILE = N_PAD // NS

W3 = 40
GW3 = E // W3
T3 = GW3 // NW
W15 = 128
GW15 = E // W15
T15_LO = GW15 // NW
T15_EXTRA = GW15 - NW * T15_LO

_MESH = plsc.VectorSubcoreMesh(core_axis_name="c", subcore_axis_name="s")
_SC_PARAMS = pltpu.CompilerParams(use_tc_tiling_on_sc=False)


def _zero_fill(ref, n16):
    def body(i, _):
        ref[pl.ds(i * 16, 16)] = jnp.zeros((16,), jnp.float32)
        return 0
    lax.fori_loop(0, n16, body, 0)


def _ring(src_hbm, sidx, didx, buf, dst_sh, gsems, ssems, cnt):
    pltpu.async_copy(src_hbm.at[sidx.at[0]], buf.at[0], gsems[0])
    pltpu.async_copy(src_hbm.at[sidx.at[1]], buf.at[1], gsems[1])

    def one(j, b):
        f = (b + 2) % 4
        pltpu.make_async_copy(src_hbm.at[sidx.at[j]], buf.at[b],
                              gsems[b]).wait()
        pltpu.async_copy(buf.at[b], dst_sh.at[didx.at[j]], ssems[b], add=True)

        @pl.when(j >= 2)
        def _():
            pltpu.make_async_copy(buf.at[f], dst_sh.at[didx.at[j - 2]],
                                  ssems[f]).wait()

        @pl.when(j + 2 < cnt)
        def _():
            pltpu.async_copy(src_hbm.at[sidx.at[j + 2]], buf.at[f], gsems[f])

    def body(k, _):
        for b in range(4):
            one(k * 4 + b, b)
        return 0
    lax.fori_loop(0, cnt // 4, body, 0)
    for j in range(4 * (cnt // 4), cnt):
        one(j, j % 4)
    for j in range(cnt - 2, cnt):
        pltpu.make_async_copy(buf.at[j % 4], dst_sh.at[didx.at[j]],
                              ssems[j % 4]).wait()


@functools.partial(
    pl.kernel,
    out_type=jax.ShapeDtypeStruct((NC, N_PAD), jnp.float32),
    mesh=_MESH,
    scratch_types=[
        pltpu.VMEM((T15_LO + 1, W15), jnp.int32),
        pltpu.VMEM((W15,), jnp.float32),
        pltpu.VMEM((ROWS_PER_TILE,), jnp.float32),
        pltpu.VMEM_SHARED((N_PAD,), jnp.float32),
    ],
    compiler_params=_SC_PARAMS,
)
def _deg_kernel(ei15_hbm, deg_hbm, didx, ones_v, zero_v, deg_sh):
    c = lax.axis_index("c")
    s = lax.axis_index("s")
    wid = c * NS + s

    def fill_ones(i, _):
        ones_v[pl.ds(i * 16, 16)] = jnp.ones((16,), jnp.float32)
        return 0
    lax.fori_loop(0, W15 // 16, fill_ones, 0)
    _zero_fill(zero_v, ROWS_PER_TILE // 16)

    pltpu.sync_copy(zero_v, deg_sh.at[pl.ds(s * ROWS_PER_TILE, ROWS_PER_TILE)])
    plsc.subcore_barrier()

    def scatter_ones(base, cnt):
        pltpu.sync_copy(ei15_hbm.at[1, pl.ds(base, cnt)],
                        didx.at[pl.ds(0, cnt)])

        def body(j, _):
            pltpu.sync_copy(ones_v, deg_sh.at[didx.at[j]], add=True)
            return 0
        lax.fori_loop(0, cnt, body, 0)

    @pl.when(wid < T15_EXTRA)
    def _():
        scatter_ones((T15_LO + 1) * wid, T15_LO + 1)

    @pl.when(wid >= T15_EXTRA)
    def _():
        scatter_ones(T15_LO * wid + T15_EXTRA, T15_LO)

    plsc.subcore_barrier()
    sl = pl.ds(s * ROWS_PER_TILE, ROWS_PER_TILE)
    pltpu.sync_copy(deg_sh.at[sl], deg_hbm.at[c, sl])


def _scale_body(deg2, x, dinv_o, xs_o):
    d = deg2[0, :N] + deg2[1, :N] + 1.0
    dv = lax.rsqrt(d)
    dinv_o[...] = dv
    xs_o[...] = x[...] * dv[:, None]


def _scale_call(deg2, x):
    return pl.pallas_call(
        _scale_body,
        out_shape=[
            jax.ShapeDtypeStruct((N,), jnp.float32),
            jax.ShapeDtypeStruct((N, D_IN), jnp.float32),
        ],
    )(deg2, x)


@functools.partial(
    pl.kernel,
    out_type=jax.ShapeDtypeStruct((NC, N_PAD, D_IN), jnp.float32),
    mesh=_MESH,
    scratch_types=[
        pltpu.VMEM((128, W3), jnp.int32),
        pltpu.VMEM((128, W3), jnp.int32),
        pltpu.VMEM((4, W3, D_IN), jnp.float32),
        pltpu.VMEM((16, D_IN), jnp.float32),
        pltpu.VMEM_SHARED((N_PAD, D_IN), jnp.float32),
        [pltpu.SemaphoreType.DMA] * 4,
        [pltpu.SemaphoreType.DMA] * 4,
    ],
    compiler_params=_SC_PARAMS,
)
def _prop_kernel(ei3_hbm, xs_hbm, p_hbm,
                 sidx, didx, buf, zrow, p_sh, gsems, ssems):
    c = lax.axis_index("c")
    s = lax.axis_index("s")
    wid = c * NS + s

    def zfill(i, _):
        zrow[i // 8, pl.ds((i % 8) * 16, 16)] = jnp.zeros((16,), jnp.float32)
        return 0
    lax.fori_loop(0, 16 * (D_IN // 16), zfill, 0)
    row0 = s * ROWS_PER_TILE

    def zbody(k, _):
        pltpu.sync_copy(zrow, p_sh.at[pl.ds(row0 + k * 16, 16)])
        return 0
    lax.fori_loop(0, ROWS_PER_TILE // 16, zbody, 0)
    plsc.subcore_barrier()

    off = 0
    for cnt in (128, T3 - 128):
        w0 = wid * T3 + off
        pltpu.sync_copy(ei3_hbm.at[0, pl.ds(w0, cnt)], sidx.at[pl.ds(0, cnt)])
        pltpu.sync_copy(ei3_hbm.at[1, pl.ds(w0, cnt)], didx.at[pl.ds(0, cnt)])
        _ring(xs_hbm, sidx, didx, buf, p_sh, gsems, ssems, cnt)
        off += cnt

    plsc.subcore_barrier()
    sl = pl.ds(row0, ROWS_PER_TILE)
    pltpu.sync_copy(p_sh.at[sl], p_hbm.at[c, sl])


def _dense_body(pA, pB, xs, dinv2, W1, b1, W2t, zs_o):
    i = pl.program_id(0)
    dv = dinv2[i]
    u = (pA[0] + pB[0] + xs[...]) * dv[:, None]
    h1 = jnp.dot(u, W1[...], preferred_element_type=jnp.float32,
                 precision=jax.lax.Precision.HIGHEST)
    h1 = jnp.maximum(h1 + b1[...], 0.0)
    z = jnp.sum(h1 * W2t[...], axis=1)
    zs_o[i] = z * dv


def _dense_call(p2, xs, dinv2, W1, b1, W2t):
    blk = 1000
    return pl.pallas_call(
        _dense_body,
        grid=(N // blk,),
        in_specs=[
            pl.BlockSpec((1, blk, D_IN), lambda i: (0, i, 0)),
            pl.BlockSpec((1, blk, D_IN), lambda i: (1, i, 0)),
            pl.BlockSpec((blk, D_IN), lambda i: (i, 0)),
            pl.BlockSpec((N // blk, blk), lambda i: (0, 0)),
            pl.BlockSpec((D_IN, D_HID), lambda i: (0, 0)),
            pl.BlockSpec((1, D_HID), lambda i: (0, 0)),
            pl.BlockSpec((1, D_HID), lambda i: (0, 0)),
        ],
        out_specs=pl.BlockSpec((N // blk, blk), lambda i: (0, 0)),
        out_shape=jax.ShapeDtypeStruct((N // blk, blk), jnp.float32),
    )(p2, p2, xs, dinv2, W1, b1, W2t)


@functools.partial(
    pl.kernel,
    out_type=jax.ShapeDtypeStruct((NC, N_PAD), jnp.float32),
    mesh=_MESH,
    scratch_types=[
        pltpu.VMEM((T15_LO + 1, W15), jnp.int32),
        pltpu.VMEM((T15_LO + 1, W15), jnp.int32),
        pltpu.VMEM((N,), jnp.float32),
        pltpu.VMEM((W15,), jnp.float32),
        pltpu.VMEM((ROWS_PER_TILE,), jnp.float32),
        pltpu.VMEM_SHARED((N_PAD,), jnp.float32),
    ],
    compiler_params=pltpu.CompilerParams(needs_layout_passes=False,
                                         use_tc_tiling_on_sc=False),
)
def _sprop_kernel(ei15_hbm, zs_hbm, o_hbm,
                  sidx, didx, zs_v, vals, zero_v, o_sh):
    c = lax.axis_index("c")
    s = lax.axis_index("s")
    wid = c * NS + s

    _zero_fill(zero_v, ROWS_PER_TILE // 16)
    pltpu.sync_copy(zero_v, o_sh.at[pl.ds(s * ROWS_PER_TILE, ROWS_PER_TILE)])
    plsc.subcore_barrier()

    pltpu.sync_copy(zs_hbm, zs_v)

    def run(base, cnt):
        pltpu.sync_copy(ei15_hbm.at[0, pl.ds(base, cnt)],
                        sidx.at[pl.ds(0, cnt)])
        pltpu.sync_copy(ei15_hbm.at[1, pl.ds(base, cnt)],
                        didx.at[pl.ds(0, cnt)])

        def body(j, _):
            for k in range(W15 // 16):
                i16 = sidx[j, pl.ds(k * 16, 16)]
                vals[pl.ds(k * 16, 16)] = plsc.load_gather(zs_v, [i16])
            pltpu.sync_copy(vals, o_sh.at[didx.at[j]], add=True)
            return 0
        lax.fori_loop(0, cnt, body, 0)

    @pl.when(wid < T15_EXTRA)
    def _():
        run((T15_LO + 1) * wid, T15_LO + 1)

    @pl.when(wid >= T15_EXTRA)
    def _():
        run(T15_LO * wid + T15_EXTRA, T15_LO)

    plsc.subcore_barrier()
    sl = pl.ds(s * ROWS_PER_TILE, ROWS_PER_TILE)
    pltpu.sync_copy(o_sh.at[sl], o_hbm.at[c, sl])


def _comb_body(o2, zs, dinv, b2, out_o):
    out_o[...] = (o2[0, :N] + o2[1, :N] + zs[...]) * dinv[...] + b2[0]


def _comb_call(o2, zs, dinv, b2):
    return pl.pallas_call(
        _comb_body,
        in_specs=[
            pl.BlockSpec(),
            pl.BlockSpec(),
            pl.BlockSpec(),
            pl.BlockSpec(memory_space=pltpu.SMEM),
        ],
        out_specs=pl.BlockSpec(),
        out_shape=jax.ShapeDtypeStruct((N,), jnp.float32),
    )(o2, zs, dinv, b2)


def kernel(x, edge_index, W1, b1, W2, b2):
    ei = edge_index.astype(jnp.int32)
    ei3 = ei.reshape(2, GW3, W3)
    ei15 = ei.reshape(2, GW15, W15)

    deg2 = _deg_kernel(ei15)
    dinv, xs = _scale_call(deg2, x)
    p2 = _prop_kernel(ei3, xs)
    zs = _dense_call(p2, xs, dinv.reshape(N // 1000, 1000), W1,
                     b1.reshape(1, D_HID), W2.reshape(1, D_HID)).reshape(N)
    o2 = _sprop_kernel(ei15, zs)
    out = _comb_call(o2, zs, dinv, b2)
    return out[:, None]


# speedup vs baseline: 2.7057x; 1.0003x over previous
import functools

import jax
import jax.numpy as jnp
from jax import lax
from jax.experimental import pallas as pl
from jax.experimental.pallas import tpu as pltpu
from jax.experimental.pallas import tpu_sc as plsc

N = 10000
D_IN = 128
D_HID = 256
E = 320000

NC, NS = 2, 16
NW = NC * NS
N_PAD = 10240
ROWS_PER_TILE = N_PAD // NS

W3 = 40
GW3 = E // W3
T3 = GW3 // NW
W15 = 128
GW15 = E // W15
T15_LO = GW15 // NW
T15_EXTRA = GW15 - NW * T15_LO

_MESH = plsc.VectorSubcoreMesh(core_axis_name="c", subcore_axis_name="s")
_SC_PARAMS = pltpu.CompilerParams(use_tc_tiling_on_sc=False)


def _zero_fill(ref, n16):
    def body(i, _):
        ref[pl.ds(i * 16, 16)] = jnp.zeros((16,), jnp.float32)
        return 0
    lax.fori_loop(0, n16, body, 0)


def _ring(src_hbm, sfn, dfn, buf, dst_sh, gsems, ssems, cnt):
    pltpu.async_copy(src_hbm.at[sfn(0)], buf.at[0], gsems[0])
    pltpu.async_copy(src_hbm.at[sfn(1)], buf.at[1], gsems[1])

    def one(j, b):
        f = (b + 2) % 4
        pltpu.make_async_copy(src_hbm.at[sfn(j)], buf.at[b],
                              gsems[b]).wait()
        pltpu.async_copy(buf.at[b], dst_sh.at[dfn(j)], ssems[b], add=True)

        @pl.when(j >= 2)
        def _():
            pltpu.make_async_copy(buf.at[f], dst_sh.at[dfn(j - 2)],
                                  ssems[f]).wait()

        @pl.when(j + 2 < cnt)
        def _():
            pltpu.async_copy(src_hbm.at[sfn(j + 2)], buf.at[f], gsems[f])

    def body(k, _):
        for b in range(4):
            one(k * 4 + b, b)
        return 0
    lax.fori_loop(0, cnt // 4, body, 0)
    for j in range(4 * (cnt // 4), cnt):
        one(j, j % 4)
    for j in range(cnt - 2, cnt):
        pltpu.make_async_copy(buf.at[j % 4], dst_sh.at[dfn(j)],
                              ssems[j % 4]).wait()


@functools.partial(
    pl.kernel,
    out_type=jax.ShapeDtypeStruct((NC, N_PAD), jnp.float32),
    mesh=_MESH,
    scratch_types=[
        pltpu.VMEM((T15_LO + 1, W15), jnp.int32),
        pltpu.VMEM((W15,), jnp.float32),
        pltpu.VMEM((ROWS_PER_TILE,), jnp.float32),
        pltpu.VMEM_SHARED((N_PAD,), jnp.float32),
    ],
    compiler_params=_SC_PARAMS,
)
def _deg_kernel(ei15_hbm, deg_hbm, didx, ones_v, zero_v, deg_sh):
    c = lax.axis_index("c")
    s = lax.axis_index("s")
    wid = c * NS + s

    def fill_ones(i, _):
        ones_v[pl.ds(i * 16, 16)] = jnp.ones((16,), jnp.float32)
        return 0
    lax.fori_loop(0, W15 // 16, fill_ones, 0)
    _zero_fill(zero_v, ROWS_PER_TILE // 16)

    pltpu.sync_copy(zero_v, deg_sh.at[pl.ds(s * ROWS_PER_TILE, ROWS_PER_TILE)])
    plsc.subcore_barrier()

    def scatter_ones(base, cnt):
        pltpu.sync_copy(ei15_hbm.at[1, pl.ds(base, cnt)],
                        didx.at[pl.ds(0, cnt)])

        def body(j, _):
            pltpu.sync_copy(ones_v, deg_sh.at[didx.at[j]], add=True)
            return 0
        lax.fori_loop(0, cnt, body, 0)

    @pl.when(wid < T15_EXTRA)
    def _():
        scatter_ones((T15_LO + 1) * wid, T15_LO + 1)

    @pl.when(wid >= T15_EXTRA)
    def _():
        scatter_ones(T15_LO * wid + T15_EXTRA, T15_LO)

    plsc.subcore_barrier()
    sl = pl.ds(s * ROWS_PER_TILE, ROWS_PER_TILE)
    pltpu.sync_copy(deg_sh.at[sl], deg_hbm.at[c, sl])


def _scale_body(deg2, x, dinv_o, xs_o):
    d = deg2[0, :N] + deg2[1, :N] + 1.0
    dv = lax.rsqrt(d)
    dinv_o[...] = dv
    xs_o[...] = x[...] * dv[:, None]


def _scale_call(deg2, x):
    return pl.pallas_call(
        _scale_body,
        out_shape=[
            jax.ShapeDtypeStruct((N,), jnp.float32),
            jax.ShapeDtypeStruct((N, D_IN), jnp.float32),
        ],
    )(deg2, x)


@functools.partial(
    pl.kernel,
    out_type=jax.ShapeDtypeStruct((NC, N_PAD, D_IN), jnp.float32),
    mesh=_MESH,
    scratch_types=[
        pltpu.VMEM((128 * W3,), jnp.int32),
        pltpu.VMEM((128 * W3,), jnp.int32),
        pltpu.VMEM((4, W3, D_IN), jnp.float32),
        pltpu.VMEM((16, D_IN), jnp.float32),
        pltpu.VMEM_SHARED((N_PAD, D_IN), jnp.float32),
        [pltpu.SemaphoreType.DMA] * 4,
        [pltpu.SemaphoreType.DMA] * 4,
    ],
    compiler_params=_SC_PARAMS,
)
def _prop_kernel(ei3_hbm, xs_hbm, p_hbm,
                 sidx, didx, buf, zrow, p_sh, gsems, ssems):
    c = lax.axis_index("c")
    s = lax.axis_index("s")
    wid = c * NS + s

    def zfill(i, _):
        zrow[i // 8, pl.ds((i % 8) * 16, 16)] = jnp.zeros((16,), jnp.float32)
        return 0
    lax.fori_loop(0, 16 * (D_IN // 16), zfill, 0)
    row0 = s * ROWS_PER_TILE

    def zbody(k, _):
        pltpu.sync_copy(zrow, p_sh.at[pl.ds(row0 + k * 16, 16)])
        return 0
    lax.fori_loop(0, ROWS_PER_TILE // 16, zbody, 0)
    plsc.subcore_barrier()

    off = 0
    for cnt in (128, T3 - 128):
        e0 = (wid * T3 + off) * W3
        pltpu.sync_copy(ei3_hbm.at[0, pl.ds(e0, cnt * W3)],
                        sidx.at[pl.ds(0, cnt * W3)])
        pltpu.sync_copy(ei3_hbm.at[1, pl.ds(e0, cnt * W3)],
                        didx.at[pl.ds(0, cnt * W3)])
        _ring(xs_hbm,
              lambda j: sidx.at[pl.ds(j * W3, W3)],
              lambda j: didx.at[pl.ds(j * W3, W3)],
              buf, p_sh, gsems, ssems, cnt)
        off += cnt

    plsc.subcore_barrier()
    sl = pl.ds(row0, ROWS_PER_TILE)
    pltpu.sync_copy(p_sh.at[sl], p_hbm.at[c, sl])


def _dense_body(pA, pB, xs, dinv2, W1, b1, W2t, zs_o):
    i = pl.program_id(0)
    dv = dinv2[i]
    u = (pA[0] + pB[0] + xs[...]) * dv[:, None]
    h1 = jnp.dot(u, W1[...], preferred_element_type=jnp.float32,
                 precision=jax.lax.Precision.HIGHEST)
    h1 = jnp.maximum(h1 + b1[...], 0.0)
    z = jnp.sum(h1 * W2t[...], axis=1)
    zs_o[i] = z * dv


def _dense_call(p2, xs, dinv2, W1, b1, W2t):
    blk = 1000
    return pl.pallas_call(
        _dense_body,
        grid=(N // blk,),
        in_specs=[
            pl.BlockSpec((1, blk, D_IN), lambda i: (0, i, 0)),
            pl.BlockSpec((1, blk, D_IN), lambda i: (1, i, 0)),
            pl.BlockSpec((blk, D_IN), lambda i: (i, 0)),
            pl.BlockSpec((N // blk, blk), lambda i: (0, 0)),
            pl.BlockSpec((D_IN, D_HID), lambda i: (0, 0)),
            pl.BlockSpec((1, D_HID), lambda i: (0, 0)),
            pl.BlockSpec((1, D_HID), lambda i: (0, 0)),
        ],
        out_specs=pl.BlockSpec((N // blk, blk), lambda i: (0, 0)),
        out_shape=jax.ShapeDtypeStruct((N // blk, blk), jnp.float32),
    )(p2, p2, xs, dinv2, W1, b1, W2t)


@functools.partial(
    pl.kernel,
    out_type=jax.ShapeDtypeStruct((NC, N_PAD), jnp.float32),
    mesh=_MESH,
    scratch_types=[
        pltpu.VMEM((T15_LO + 1, W15), jnp.int32),
        pltpu.VMEM((T15_LO + 1, W15), jnp.int32),
        pltpu.VMEM((N,), jnp.float32),
        pltpu.VMEM((W15,), jnp.float32),
        pltpu.VMEM((ROWS_PER_TILE,), jnp.float32),
        pltpu.VMEM_SHARED((N_PAD,), jnp.float32),
    ],
    compiler_params=pltpu.CompilerParams(needs_layout_passes=False,
                                         use_tc_tiling_on_sc=False),
)
def _sprop_kernel(ei15_hbm, zs_hbm, o_hbm,
                  sidx, didx, zs_v, vals, zero_v, o_sh):
    c = lax.axis_index("c")
    s = lax.axis_index("s")
    wid = c * NS + s

    _zero_fill(zero_v, ROWS_PER_TILE // 16)
    pltpu.sync_copy(zero_v, o_sh.at[pl.ds(s * ROWS_PER_TILE, ROWS_PER_TILE)])
    plsc.subcore_barrier()

    pltpu.sync_copy(zs_hbm, zs_v)

    def run(base, cnt):
        pltpu.sync_copy(ei15_hbm.at[0, pl.ds(base, cnt)],
                        sidx.at[pl.ds(0, cnt)])
        pltpu.sync_copy(ei15_hbm.at[1, pl.ds(base, cnt)],
                        didx.at[pl.ds(0, cnt)])

        def body(j, _):
            for k in range(W15 // 16):
                i16 = sidx[j, pl.ds(k * 16, 16)]
                vals[pl.ds(k * 16, 16)] = plsc.load_gather(zs_v, [i16])
            pltpu.sync_copy(vals, o_sh.at[didx.at[j]], add=True)
            return 0
        lax.fori_loop(0, cnt, body, 0)

    @pl.when(wid < T15_EXTRA)
    def _():
        run((T15_LO + 1) * wid, T15_LO + 1)

    @pl.when(wid >= T15_EXTRA)
    def _():
        run(T15_LO * wid + T15_EXTRA, T15_LO)

    plsc.subcore_barrier()
    sl = pl.ds(s * ROWS_PER_TILE, ROWS_PER_TILE)
    pltpu.sync_copy(o_sh.at[sl], o_hbm.at[c, sl])


def _comb_body(o2, zs, dinv, b2, out_o):
    out_o[...] = (o2[0, :N] + o2[1, :N] + zs[...]) * dinv[...] + b2[0]


def _comb_call(o2, zs, dinv, b2):
    return pl.pallas_call(
        _comb_body,
        in_specs=[
            pl.BlockSpec(),
            pl.BlockSpec(),
            pl.BlockSpec(),
            pl.BlockSpec(memory_space=pltpu.SMEM),
        ],
        out_specs=pl.BlockSpec(),
        out_shape=jax.ShapeDtypeStruct((N,), jnp.float32),
    )(o2, zs, dinv, b2)


def kernel(x, edge_index, W1, b1, W2, b2):
    ei = edge_index.astype(jnp.int32)
    ei15 = ei.reshape(2, GW15, W15)

    deg2 = _deg_kernel(ei15)
    dinv, xs = _scale_call(deg2, x)
    p2 = _prop_kernel(ei, xs)
    zs = _dense_call(p2, xs, dinv.reshape(N // 1000, 1000), W1,
                     b1.reshape(1, D_HID), W2.reshape(1, D_HID)).reshape(N)
    o2 = _sprop_kernel(ei15, zs)
    out = _comb_call(o2, zs, dinv, b2)
    return out[:, None]


# speedup vs baseline: 2.8198x; 1.0422x over previous
import functools

import jax
import jax.numpy as jnp
from jax import lax
from jax.experimental import pallas as pl
from jax.experimental.pallas import tpu as pltpu
from jax.experimental.pallas import tpu_sc as plsc

N = 10000
D_IN = 128
D_HID = 256
E = 320000

NC, NS = 2, 16
NW = NC * NS
N_PAD = 10240
ROWS_PER_TILE = N_PAD // NS

W3 = 40
GW3 = E // W3
T3 = GW3 // NW
W15 = 128
GW15 = E // W15
T15_LO = GW15 // NW
T15_EXTRA = GW15 - NW * T15_LO

_MESH = plsc.VectorSubcoreMesh(core_axis_name="c", subcore_axis_name="s")
_SC_PARAMS = pltpu.CompilerParams(use_tc_tiling_on_sc=False)


def _zero_fill(ref, n16):
    def body(i, _):
        ref[pl.ds(i * 16, 16)] = jnp.zeros((16,), jnp.float32)
        return 0
    lax.fori_loop(0, n16, body, 0)


def _ring(src_hbm, sfn, dfn, buf, dst_sh, gsems, ssems, cnt):
    pltpu.async_copy(src_hbm.at[sfn(0)], buf.at[0], gsems[0])
    pltpu.async_copy(src_hbm.at[sfn(1)], buf.at[1], gsems[1])

    def one(j, b):
        f = (b + 2) % 4
        pltpu.make_async_copy(src_hbm.at[sfn(j)], buf.at[b],
                              gsems[b]).wait()
        pltpu.async_copy(buf.at[b], dst_sh.at[dfn(j)], ssems[b], add=True)

        @pl.when(j >= 2)
        def _():
            pltpu.make_async_copy(buf.at[f], dst_sh.at[dfn(j - 2)],
                                  ssems[f]).wait()

        @pl.when(j + 2 < cnt)
        def _():
            pltpu.async_copy(src_hbm.at[sfn(j + 2)], buf.at[f], gsems[f])

    def body(k, _):
        for b in range(4):
            one(k * 4 + b, b)
        return 0
    lax.fori_loop(0, cnt // 4, body, 0)
    for j in range(4 * (cnt // 4), cnt):
        one(j, j % 4)
    for j in range(cnt - 2, cnt):
        pltpu.make_async_copy(buf.at[j % 4], dst_sh.at[dfn(j)],
                              ssems[j % 4]).wait()


@functools.partial(
    pl.kernel,
    out_type=jax.ShapeDtypeStruct((NC, N_PAD), jnp.float32),
    mesh=_MESH,
    scratch_types=[
        pltpu.VMEM((T15_LO + 1, W15), jnp.int32),
        pltpu.VMEM((W15,), jnp.float32),
        pltpu.VMEM((ROWS_PER_TILE,), jnp.float32),
        pltpu.VMEM_SHARED((N_PAD,), jnp.float32),
        [pltpu.SemaphoreType.DMA] * 2,
    ],
    compiler_params=_SC_PARAMS,
)
def _deg_kernel(ei15_hbm, deg_hbm, didx, ones_v, zero_v, deg_sh, ssems):
    c = lax.axis_index("c")
    s = lax.axis_index("s")
    wid = c * NS + s

    def fill_ones(i, _):
        ones_v[pl.ds(i * 16, 16)] = jnp.ones((16,), jnp.float32)
        return 0
    lax.fori_loop(0, W15 // 16, fill_ones, 0)
    _zero_fill(zero_v, ROWS_PER_TILE // 16)

    pltpu.sync_copy(zero_v, deg_sh.at[pl.ds(s * ROWS_PER_TILE, ROWS_PER_TILE)])
    plsc.subcore_barrier()

    def scatter_ones(base, cnt):
        pltpu.sync_copy(ei15_hbm.at[1, pl.ds(base, cnt)],
                        didx.at[pl.ds(0, cnt)])

        def one(j, b):
            @pl.when(j >= 2)
            def _():
                pltpu.make_async_copy(ones_v, deg_sh.at[didx.at[j - 2]],
                                      ssems[b]).wait()
            pltpu.async_copy(ones_v, deg_sh.at[didx.at[j]], ssems[b],
                             add=True)

        def body(k, _):
            for b in range(2):
                one(k * 2 + b, b)
            return 0
        lax.fori_loop(0, cnt // 2, body, 0)
        for j in range(2 * (cnt // 2), cnt):
            one(j, j % 2)
        for j in range(cnt - 2, cnt):
            pltpu.make_async_copy(ones_v, deg_sh.at[didx.at[j]],
                                  ssems[j % 2]).wait()

    @pl.when(wid < T15_EXTRA)
    def _():
        scatter_ones((T15_LO + 1) * wid, T15_LO + 1)

    @pl.when(wid >= T15_EXTRA)
    def _():
        scatter_ones(T15_LO * wid + T15_EXTRA, T15_LO)

    plsc.subcore_barrier()
    sl = pl.ds(s * ROWS_PER_TILE, ROWS_PER_TILE)
    pltpu.sync_copy(deg_sh.at[sl], deg_hbm.at[c, sl])


def _scale_body(deg2, x, dinv_o, xs_o):
    d = deg2[0, :N] + deg2[1, :N] + 1.0
    dv = lax.rsqrt(d)
    dinv_o[...] = dv
    xs_o[...] = x[...] * dv[:, None]


def _scale_call(deg2, x):
    return pl.pallas_call(
        _scale_body,
        out_shape=[
            jax.ShapeDtypeStruct((N,), jnp.float32),
            jax.ShapeDtypeStruct((N, D_IN), jnp.float32),
        ],
    )(deg2, x)


@functools.partial(
    pl.kernel,
    out_type=jax.ShapeDtypeStruct((NC, N_PAD, D_IN), jnp.float32),
    mesh=_MESH,
    scratch_types=[
        pltpu.VMEM((128 * W3,), jnp.int32),
        pltpu.VMEM((128 * W3,), jnp.int32),
        pltpu.VMEM((4, W3, D_IN), jnp.float32),
        pltpu.VMEM((16, D_IN), jnp.float32),
        pltpu.VMEM_SHARED((N_PAD, D_IN), jnp.float32),
        [pltpu.SemaphoreType.DMA] * 4,
        [pltpu.SemaphoreType.DMA] * 4,
    ],
    compiler_params=_SC_PARAMS,
)
def _prop_kernel(ei3_hbm, xs_hbm, p_hbm,
                 sidx, didx, buf, zrow, p_sh, gsems, ssems):
    c = lax.axis_index("c")
    s = lax.axis_index("s")
    wid = c * NS + s

    def zfill(i, _):
        zrow[i // 8, pl.ds((i % 8) * 16, 16)] = jnp.zeros((16,), jnp.float32)
        return 0
    lax.fori_loop(0, 16 * (D_IN // 16), zfill, 0)
    row0 = s * ROWS_PER_TILE

    def zbody(k, _):
        pltpu.sync_copy(zrow, p_sh.at[pl.ds(row0 + k * 16, 16)])
        return 0
    lax.fori_loop(0, ROWS_PER_TILE // 16, zbody, 0)
    plsc.subcore_barrier()

    off = 0
    for cnt in (128, T3 - 128):
        e0 = (wid * T3 + off) * W3
        pltpu.sync_copy(ei3_hbm.at[0, pl.ds(e0, cnt * W3)],
                        sidx.at[pl.ds(0, cnt * W3)])
        pltpu.sync_copy(ei3_hbm.at[1, pl.ds(e0, cnt * W3)],
                        didx.at[pl.ds(0, cnt * W3)])
        _ring(xs_hbm,
              lambda j: sidx.at[pl.ds(j * W3, W3)],
              lambda j: didx.at[pl.ds(j * W3, W3)],
              buf, p_sh, gsems, ssems, cnt)
        off += cnt

    plsc.subcore_barrier()
    sl = pl.ds(row0, ROWS_PER_TILE)
    pltpu.sync_copy(p_sh.at[sl], p_hbm.at[c, sl])


def _dense_body(pA, pB, xs, dinv2, W1, b1, W2t, zs_o):
    i = pl.program_id(0)
    dv = dinv2[i]
    u = (pA[0] + pB[0] + xs[...]) * dv[:, None]
    h1 = jnp.dot(u, W1[...], preferred_element_type=jnp.float32,
                 precision=jax.lax.Precision.HIGHEST)
    h1 = jnp.maximum(h1 + b1[...], 0.0)
    z = jnp.sum(h1 * W2t[...], axis=1)
    zs_o[i] = z * dv


def _dense_call(p2, xs, dinv2, W1, b1, W2t):
    blk = 1000
    return pl.pallas_call(
        _dense_body,
        grid=(N // blk,),
        in_specs=[
            pl.BlockSpec((1, blk, D_IN), lambda i: (0, i, 0)),
            pl.BlockSpec((1, blk, D_IN), lambda i: (1, i, 0)),
            pl.BlockSpec((blk, D_IN), lambda i: (i, 0)),
            pl.BlockSpec((N // blk, blk), lambda i: (0, 0)),
            pl.BlockSpec((D_IN, D_HID), lambda i: (0, 0)),
            pl.BlockSpec((1, D_HID), lambda i: (0, 0)),
            pl.BlockSpec((1, D_HID), lambda i: (0, 0)),
        ],
        out_specs=pl.BlockSpec((N // blk, blk), lambda i: (0, 0)),
        out_shape=jax.ShapeDtypeStruct((N // blk, blk), jnp.float32),
    )(p2, p2, xs, dinv2, W1, b1, W2t)


@functools.partial(
    pl.kernel,
    out_type=jax.ShapeDtypeStruct((NC, N_PAD), jnp.float32),
    mesh=_MESH,
    scratch_types=[
        pltpu.VMEM((T15_LO + 1, W15), jnp.int32),
        pltpu.VMEM((T15_LO + 1, W15), jnp.int32),
        pltpu.VMEM((N,), jnp.float32),
        pltpu.VMEM((2, W15), jnp.float32),
        pltpu.VMEM((ROWS_PER_TILE,), jnp.float32),
        pltpu.VMEM_SHARED((N_PAD,), jnp.float32),
        [pltpu.SemaphoreType.DMA] * 2,
    ],
    compiler_params=pltpu.CompilerParams(needs_layout_passes=False,
                                         use_tc_tiling_on_sc=False),
)
def _sprop_kernel(ei15_hbm, zs_hbm, o_hbm,
                  sidx, didx, zs_v, vals, zero_v, o_sh, ssems):
    c = lax.axis_index("c")
    s = lax.axis_index("s")
    wid = c * NS + s

    _zero_fill(zero_v, ROWS_PER_TILE // 16)
    pltpu.sync_copy(zero_v, o_sh.at[pl.ds(s * ROWS_PER_TILE, ROWS_PER_TILE)])
    plsc.subcore_barrier()

    pltpu.sync_copy(zs_hbm, zs_v)

    def run(base, cnt):
        pltpu.sync_copy(ei15_hbm.at[0, pl.ds(base, cnt)],
                        sidx.at[pl.ds(0, cnt)])
        pltpu.sync_copy(ei15_hbm.at[1, pl.ds(base, cnt)],
                        didx.at[pl.ds(0, cnt)])

        def one(j, b):
            @pl.when(j >= 2)
            def _():
                pltpu.make_async_copy(vals.at[b], o_sh.at[didx.at[j - 2]],
                                      ssems[b]).wait()
            for k in range(W15 // 16):
                i16 = sidx[j, pl.ds(k * 16, 16)]
                vals[b, pl.ds(k * 16, 16)] = plsc.load_gather(zs_v, [i16])
            pltpu.async_copy(vals.at[b], o_sh.at[didx.at[j]], ssems[b],
                             add=True)

        def body(k, _):
            for b in range(2):
                one(k * 2 + b, b)
            return 0
        lax.fori_loop(0, cnt // 2, body, 0)
        for j in range(2 * (cnt // 2), cnt):
            one(j, j % 2)
        for j in range(cnt - 2, cnt):
            pltpu.make_async_copy(vals.at[j % 2], o_sh.at[didx.at[j]],
                                  ssems[j % 2]).wait()

    @pl.when(wid < T15_EXTRA)
    def _():
        run((T15_LO + 1) * wid, T15_LO + 1)

    @pl.when(wid >= T15_EXTRA)
    def _():
        run(T15_LO * wid + T15_EXTRA, T15_LO)

    plsc.subcore_barrier()
    sl = pl.ds(s * ROWS_PER_TILE, ROWS_PER_TILE)
    pltpu.sync_copy(o_sh.at[sl], o_hbm.at[c, sl])


def _comb_body(o2, zs, dinv, b2, out_o):
    out_o[...] = (o2[0, :N] + o2[1, :N] + zs[...]) * dinv[...] + b2[0]


def _comb_call(o2, zs, dinv, b2):
    return pl.pallas_call(
        _comb_body,
        in_specs=[
            pl.BlockSpec(),
            pl.BlockSpec(),
            pl.BlockSpec(),
            pl.BlockSpec(memory_space=pltpu.SMEM),
        ],
        out_specs=pl.BlockSpec(),
        out_shape=jax.ShapeDtypeStruct((N,), jnp.float32),
    )(o2, zs, dinv, b2)


def kernel(x, edge_index, W1, b1, W2, b2):
    ei = edge_index.astype(jnp.int32)
    ei15 = ei.reshape(2, GW15, W15)

    deg2 = _deg_kernel(ei15)
    dinv, xs = _scale_call(deg2, x)
    p2 = _prop_kernel(ei, xs)
    zs = _dense_call(p2, xs, dinv.reshape(N // 1000, 1000), W1,
                     b1.reshape(1, D_HID), W2.reshape(1, D_HID)).reshape(N)
    o2 = _sprop_kernel(ei15, zs)
    out = _comb_call(o2, zs, dinv, b2)
    return out[:, None]
